# baseline (speedup 1.0000x reference)
import jax
import jax.numpy as jnp
from jax import lax
from jax.experimental import pallas as pl
from jax.experimental.pallas import tpu as pltpu
from jax.experimental.pallas import tpu_sc as plsc
import functools

N_NODES = 10000
N_PAD = 10240
D_FEAT = 256
DH = 128
N_EDGES = 160000
NC = 2
NS = 16
EPT = N_EDGES // NS
B = 80
NCH = EPT // B
NBUF = 4
NCH_MAIN = NCH - 1
RPT = N_PAD // NS
RB = 32
NRB = RPT // RB
N_TAIL = N_NODES % RB


def _body(news_ref, ed_ref, out_ref,
          acc, ws,
          rows0, rows1, rows2, rows3,
          ed0, ed1, ed2, ed3,
          ix0, ix1, ix2, ix3,
          wa0, wa1, wa2, wa3,
          blk_v, ws_v,
          sem0, sem1, sem2, sem3):
    c = lax.axis_index("c")
    s = lax.axis_index("s")
    zero16 = jnp.zeros((16,), jnp.float32)
    rows = (rows0, rows1, rows2, rows3)
    edb = (ed0, ed1, ed2, ed3)
    idxb = (ix0, ix1, ix2, ix3)
    wab = (wa0, wa1, wa2, wa3)
    sems = (sem0, sem1, sem2, sem3)

    def fetch_ed(j, p):
        pltpu.async_copy(ed_ref.at[s, j], edb[p], sems[p])

    def wait_ed(j, p):
        pltpu.make_async_copy(ed_ref.at[s, j], edb[p], sems[p]).wait()

    def prep(j, p):
        eb, ib, wa = edb[p], idxb[p], wab[p]
        for k in range(B // 16):
            sl = pl.ds(k * 16, 16)
            ib[sl] = eb[2, sl] * 2 + c
            wa[sl] = lax.bitcast_convert_type(eb[1, sl], jnp.float32)
        pltpu.async_copy(news_ref.at[idxb[p]], rows[p], sems[p])

    def wait_rows(p):
        pltpu.make_async_copy(news_ref.at[idxb[p]], rows[p], sems[p]).wait()

    def scatter(p):
        idx = edb[p].at[0]
        pltpu.async_copy(rows[p], acc.at[idx], sems[p], add=True)
        pltpu.async_copy(wab[p].at[pl.ds(0, B)], ws.at[idx], sems[p], add=True)

    def wait_scatter(p):
        idx = edb[p].at[0]
        pltpu.make_async_copy(rows[p], acc.at[idx], sems[p]).wait()
        pltpu.make_async_copy(wab[p].at[pl.ds(0, B)], ws.at[idx],
                              sems[p]).wait()

    def scale(p):
        rows_p, wa = rows[p], wab[p]

        def body(e, _):
            a16 = wa[pl.ds(e, 16)]
            av = jnp.broadcast_to(a16[0], (16,))
            for v in range(DH // 16):
                sl = pl.ds(v * 16, 16)
                rows_p[e, sl] = rows_p[e, sl] * av
            return 0
        lax.fori_loop(0, B, body, 0)

    def zrow(i, _):
        for v in range(DH // 16):
            blk_v[i, pl.ds(v * 16, 16)] = zero16
        return 0
    lax.fori_loop(0, RB, zrow, 0)
    r0t = s * RPT
    for k in range(NRB):
        pltpu.sync_copy(blk_v, acc.at[pl.ds(r0t + k * RB, RB)])
    for k in range(RPT // DH):
        pltpu.sync_copy(blk_v.at[0], ws.at[pl.ds(r0t + k * DH, DH)])

    plsc.subcore_barrier()

    fetch_ed(0, 0)
    fetch_ed(1, 1)
    wait_ed(0, 0)
    prep(0, 0)

    def outer(j4, _):
        for p in range(NBUF):
            j = j4 * NBUF + p
            p1 = (p + 1) % NBUF
            p2 = (p + 2) % NBUF

            @pl.when(j >= 2)
            def _():
                wait_scatter(p2)

            @pl.when(j + 2 < NCH)
            def _():
                fetch_ed(j + 2, p2)

            @pl.when(j + 1 < NCH)
            def _():
                wait_ed(j + 1, p1)
                prep(j + 1, p1)

            wait_rows(p)
            scale(p)
            scatter(p)
        return 0
    lax.fori_loop(0, NCH_MAIN // NBUF, outer, 0)

    for jt in range(NCH_MAIN, NCH):
        p = jt % NBUF
        wait_scatter((p + 2) % NBUF)
        wait_rows(p)
        scale(p)
        scatter(p)
    wait_scatter((NCH - 2) % NBUF)
    wait_scatter((NCH - 1) % NBUF)

    plsc.subcore_barrier()

    for k in range(NRB):
        r0 = r0t + k * RB
        pltpu.sync_copy(acc.at[pl.ds(r0, RB)], blk_v)
        pltpu.sync_copy(ws.at[pl.ds(r0, RB)], ws_v.at[pl.ds(0, RB)])

        def fin(r, _):
            w16 = ws_v[pl.ds(r, 16)]
            w = jnp.broadcast_to(w16[0], (16,))
            iv = 1.0 / jnp.maximum(w, 1e-9)
            for v in range(DH // 16):
                sl = pl.ds(v * 16, 16)
                blk_v[r, sl] = blk_v[r, sl] * iv
            return 0
        lax.fori_loop(0, RB, fin, 0)

        @pl.when(r0 + RB <= N_NODES)
        def _():
            pltpu.sync_copy(blk_v, out_ref.at[pl.ds(r0, RB), pl.ds(c * DH, DH)])

        @pl.when(jnp.logical_and(r0 < N_NODES, r0 + RB > N_NODES))
        def _():
            pltpu.sync_copy(blk_v.at[pl.ds(0, N_TAIL)],
                            out_ref.at[pl.ds(r0, N_TAIL), pl.ds(c * DH, DH)])


@functools.partial(jax.jit, static_argnums=())
def _run(news_flat, edata):
    mesh = plsc.VectorSubcoreMesh(core_axis_name="c", subcore_axis_name="s",
                                  num_cores=NC, num_subcores=NS)
    f = pl.kernel(
        _body,
        out_type=jax.ShapeDtypeStruct((N_NODES, D_FEAT), jnp.float32),
        mesh=mesh,
        scratch_types=[
            pltpu.VMEM_SHARED((N_PAD, DH), jnp.float32),
            pltpu.VMEM_SHARED((N_PAD,), jnp.float32),
            pltpu.VMEM((B, DH), jnp.float32),
            pltpu.VMEM((B, DH), jnp.float32),
            pltpu.VMEM((B, DH), jnp.float32),
            pltpu.VMEM((B, DH), jnp.float32),
            pltpu.VMEM((3, B), jnp.int32),
            pltpu.VMEM((3, B), jnp.int32),
            pltpu.VMEM((3, B), jnp.int32),
            pltpu.VMEM((3, B), jnp.int32),
            pltpu.VMEM((B,), jnp.int32),
            pltpu.VMEM((B,), jnp.int32),
            pltpu.VMEM((B,), jnp.int32),
            pltpu.VMEM((B,), jnp.int32),
            pltpu.VMEM((B + 16,), jnp.float32),
            pltpu.VMEM((B + 16,), jnp.float32),
            pltpu.VMEM((B + 16,), jnp.float32),
            pltpu.VMEM((B + 16,), jnp.float32),
            pltpu.VMEM((RB, DH), jnp.float32),
            pltpu.VMEM((RB + 16,), jnp.float32),
            pltpu.SemaphoreType.DMA,
            pltpu.SemaphoreType.DMA,
            pltpu.SemaphoreType.DMA,
            pltpu.SemaphoreType.DMA,
        ],
    )
    return f(news_flat, edata)


def kernel(news_x, edge_index, edge_attr, num_companies):
    del num_companies
    news_flat = news_x.reshape(2 * N_NODES, DH)
    ei = edge_index.astype(jnp.int32)
    dst_r = ei[1].reshape(NS, NCH, B)
    src_r = ei[0].reshape(NS, NCH, B)
    attr_r = lax.bitcast_convert_type(edge_attr, jnp.int32).reshape(NS, NCH, B)
    edata = jnp.stack([dst_r, attr_r, src_r], axis=2)
    return _run(news_flat, edata)


# speedup vs baseline: 1.1598x; 1.1598x over previous
import jax
import jax.numpy as jnp
from jax import lax
from jax.experimental import pallas as pl
from jax.experimental.pallas import tpu as pltpu
from jax.experimental.pallas import tpu_sc as plsc
import functools

N_NODES = 10000
N_PAD = 10240
D_FEAT = 256
DH = 128
N_EDGES = 160000
NC = 2
NS = 16
EPT = N_EDGES // NS
B = 80
NCH = EPT // B
NBUF = 4
NCH_MAIN = NCH - 1
RPT = N_PAD // NS
RB = 32
NRB = RPT // RB
N_TAIL = N_NODES % RB


def _body(news_ref, src_ref, dst_ref, attr_ref, out_ref,
          acc, ws,
          rows0, rows1, rows2, rows3,
          sb0, sb1, sb2, sb3,
          db0, db1, db2, db3,
          wa0, wa1, wa2, wa3,
          blk_v, ws_v,
          sem0, sem1, sem2, sem3):
    c = lax.axis_index("c")
    s = lax.axis_index("s")
    zero16 = jnp.zeros((16,), jnp.float32)
    rows = (rows0, rows1, rows2, rows3)
    srcb = (sb0, sb1, sb2, sb3)
    dstb = (db0, db1, db2, db3)
    wab = (wa0, wa1, wa2, wa3)
    sems = (sem0, sem1, sem2, sem3)

    e0 = s * EPT

    def fetch_ed(j, p):
        pltpu.async_copy(src_ref.at[s, j], srcb[p], sems[p])
        pltpu.async_copy(dst_ref.at[s, j], dstb[p], sems[p])
        pltpu.async_copy(attr_ref.at[pl.ds(e0 + j * B, B)],
                         wab[p].at[pl.ds(0, B)], sems[p])

    def wait_ed(j, p):
        pltpu.make_async_copy(src_ref.at[s, j], srcb[p], sems[p]).wait()
        pltpu.make_async_copy(dst_ref.at[s, j], dstb[p], sems[p]).wait()
        pltpu.make_async_copy(attr_ref.at[pl.ds(e0 + j * B, B)],
                              wab[p].at[pl.ds(0, B)], sems[p]).wait()

    def prep(j, p):
        sb = srcb[p]
        for k in range(B // 16):
            sl = pl.ds(k * 16, 16)
            sb[sl] = sb[sl] * 2 + c
        pltpu.async_copy(news_ref.at[srcb[p]], rows[p], sems[p])

    def wait_rows(p):
        pltpu.make_async_copy(news_ref.at[srcb[p]], rows[p], sems[p]).wait()

    def scatter(p):
        pltpu.async_copy(rows[p], acc.at[dstb[p]], sems[p], add=True)
        pltpu.async_copy(wab[p].at[pl.ds(0, B)], ws.at[dstb[p]],
                         sems[p], add=True)

    def wait_scatter(p):
        pltpu.make_async_copy(rows[p], acc.at[dstb[p]], sems[p]).wait()
        pltpu.make_async_copy(wab[p].at[pl.ds(0, B)], ws.at[dstb[p]],
                              sems[p]).wait()

    def scale(p):
        rows_p, wa = rows[p], wab[p]

        def grp(g, _):
            base = g * 16
            a16 = wa[pl.ds(base, 16)]
            for i in range(16):
                av = jnp.broadcast_to(a16[i], (16,))
                for v in range(DH // 16):
                    sl = pl.ds(v * 16, 16)
                    rows_p[base + i, sl] = rows_p[base + i, sl] * av
            return 0
        lax.fori_loop(0, B // 16, grp, 0)

    def zrow(i, _):
        for v in range(DH // 16):
            blk_v[i, pl.ds(v * 16, 16)] = zero16
        return 0
    lax.fori_loop(0, RB, zrow, 0)
    r0t = s * RPT
    for k in range(NRB):
        pltpu.sync_copy(blk_v, acc.at[pl.ds(r0t + k * RB, RB)])
    for k in range(RPT // DH):
        pltpu.sync_copy(blk_v.at[0], ws.at[pl.ds(r0t + k * DH, DH)])

    plsc.subcore_barrier()

    fetch_ed(0, 0)
    fetch_ed(1, 1)
    wait_ed(0, 0)
    prep(0, 0)

    def outer(j4, _):
        for p in range(NBUF):
            j = j4 * NBUF + p
            p1 = (p + 1) % NBUF
            p2 = (p + 2) % NBUF

            @pl.when(j >= 2)
            def _():
                wait_scatter(p2)

            @pl.when(j + 2 < NCH)
            def _():
                fetch_ed(j + 2, p2)

            @pl.when(j + 1 < NCH)
            def _():
                wait_ed(j + 1, p1)
                prep(j + 1, p1)

            wait_rows(p)
            scale(p)
            scatter(p)
        return 0
    lax.fori_loop(0, NCH_MAIN // NBUF, outer, 0)

    for jt in range(NCH_MAIN, NCH):
        p = jt % NBUF
        wait_scatter((p + 2) % NBUF)
        wait_rows(p)
        scale(p)
        scatter(p)
    wait_scatter((NCH - 2) % NBUF)
    wait_scatter((NCH - 1) % NBUF)

    plsc.subcore_barrier()

    for k in range(NRB):
        r0 = r0t + k * RB
        pltpu.sync_copy(acc.at[pl.ds(r0, RB)], blk_v)
        pltpu.sync_copy(ws.at[pl.ds(r0, RB)], ws_v.at[pl.ds(0, RB)])

        def fin(r, _):
            w16 = ws_v[pl.ds(r, 16)]
            w = jnp.broadcast_to(w16[0], (16,))
            iv = 1.0 / jnp.maximum(w, 1e-9)
            for v in range(DH // 16):
                sl = pl.ds(v * 16, 16)
                blk_v[r, sl] = blk_v[r, sl] * iv
            return 0
        lax.fori_loop(0, RB, fin, 0)

        @pl.when(r0 + RB <= N_NODES)
        def _():
            pltpu.sync_copy(blk_v, out_ref.at[pl.ds(r0, RB), pl.ds(c * DH, DH)])

        @pl.when(jnp.logical_and(r0 < N_NODES, r0 + RB > N_NODES))
        def _():
            pltpu.sync_copy(blk_v.at[pl.ds(0, N_TAIL)],
                            out_ref.at[pl.ds(r0, N_TAIL), pl.ds(c * DH, DH)])


@functools.partial(jax.jit, static_argnums=())
def _run(news_flat, src_r, dst_r, attr_r):
    mesh = plsc.VectorSubcoreMesh(core_axis_name="c", subcore_axis_name="s",
                                  num_cores=NC, num_subcores=NS)
    f = pl.kernel(
        _body,
        out_type=jax.ShapeDtypeStruct((N_NODES, D_FEAT), jnp.float32),
        mesh=mesh,
        scratch_types=[
            pltpu.VMEM_SHARED((N_PAD, DH), jnp.float32),
            pltpu.VMEM_SHARED((N_PAD,), jnp.float32),
            pltpu.VMEM((B, DH), jnp.float32),
            pltpu.VMEM((B, DH), jnp.float32),
            pltpu.VMEM((B, DH), jnp.float32),
            pltpu.VMEM((B, DH), jnp.float32),
            pltpu.VMEM((B,), jnp.int32),
            pltpu.VMEM((B,), jnp.int32),
            pltpu.VMEM((B,), jnp.int32),
            pltpu.VMEM((B,), jnp.int32),
            pltpu.VMEM((B,), jnp.int32),
            pltpu.VMEM((B,), jnp.int32),
            pltpu.VMEM((B,), jnp.int32),
            pltpu.VMEM((B,), jnp.int32),
            pltpu.VMEM((B + 16,), jnp.float32),
            pltpu.VMEM((B + 16,), jnp.float32),
            pltpu.VMEM((B + 16,), jnp.float32),
            pltpu.VMEM((B + 16,), jnp.float32),
            pltpu.VMEM((RB, DH), jnp.float32),
            pltpu.VMEM((RB + 16,), jnp.float32),
            pltpu.SemaphoreType.DMA,
            pltpu.SemaphoreType.DMA,
            pltpu.SemaphoreType.DMA,
            pltpu.SemaphoreType.DMA,
        ],
    )
    return f(news_flat, src_r, dst_r, attr_r)


def kernel(news_x, edge_index, edge_attr, num_companies):
    del num_companies
    news_flat = news_x.reshape(2 * N_NODES, DH)
    ei = edge_index.astype(jnp.int32)
    src_r = ei[0].reshape(NS, NCH, B)
    dst_r = ei[1].reshape(NS, NCH, B)
    return _run(news_flat, src_r, dst_r, edge_attr)
